# baseline (speedup 1.0000x reference)
import functools

import jax
import jax.numpy as jnp
import numpy as np
from jax.experimental import pallas as pl
from jax.experimental.pallas import tpu as pltpu
from jax.experimental.pallas import tpu_sc as plsc

B, N, M, C, H, K = 4, 2048, 4096, 256, 8, 8
RADIUS = 0.2
DH = C // H
FF = 4 * C
KP = K + 1
BN = B * N
BM = B * M


def _proj_kv_body(kvf_ref, kvx_ref, wk_ref, wv_ref, wpk_ref, ck_ref, bv_ref,
                  kout_ref, vout_ref):
    kvf = kvf_ref[...]
    kvx = kvx_ref[...]
    kout_ref[...] = (jnp.dot(kvf, wk_ref[...], preferred_element_type=jnp.float32)
                     + jnp.dot(kvx, wpk_ref[...], preferred_element_type=jnp.float32)
                     + ck_ref[...])
    vout_ref[...] = (jnp.dot(kvf, wv_ref[...], preferred_element_type=jnp.float32)
                     + bv_ref[...])


def _proj_kv(kvf2, kvx2, Wk, Wv, Wpk, ck, bv, interpret=False):
    T = 2048
    grid = (BM // T,)
    return pl.pallas_call(
        _proj_kv_body,
        grid=grid,
        in_specs=[
            pl.BlockSpec((T, C), lambda i: (i, 0)),
            pl.BlockSpec((T, 3), lambda i: (i, 0)),
            pl.BlockSpec((C, C), lambda i: (0, 0)),
            pl.BlockSpec((C, C), lambda i: (0, 0)),
            pl.BlockSpec((3, C), lambda i: (0, 0)),
            pl.BlockSpec((1, C), lambda i: (0, 0)),
            pl.BlockSpec((1, C), lambda i: (0, 0)),
        ],
        out_specs=[
            pl.BlockSpec((T, C), lambda i: (i, 0)),
            pl.BlockSpec((T, C), lambda i: (i, 0)),
        ],
        out_shape=[
            jax.ShapeDtypeStruct((BM, C), jnp.float32),
            jax.ShapeDtypeStruct((BM, C), jnp.float32),
        ],
        interpret=interpret,
    )(kvf2, kvx2, Wk, Wv, Wpk, ck, bv)


_TN = 128


def _topk_body(qx_ref, kxt_ref, idx_ref, valid_ref):
    b = pl.program_id(0)
    qx = qx_ref[0]
    kxt = kxt_ref[0]
    qn = jnp.sum(qx * qx, axis=1, keepdims=True)
    kn = jnp.sum(kxt * kxt, axis=0, keepdims=True)
    dot = jnp.dot(qx.astype(jnp.bfloat16), kxt.astype(jnp.bfloat16),
                  preferred_element_type=jnp.float32)
    d2 = qn + kn - 2.0 * dot
    dist = jnp.sqrt(jnp.maximum(d2, 1e-12))
    work = jnp.where(dist <= RADIUS, dist, jnp.inf)
    base = b * M
    MH = M // 2
    lo = work[:, :MH]
    hi = work[:, MH:]
    le = lo <= hi
    iota2 = jax.lax.broadcasted_iota(jnp.int32, (_TN, MH), 1)
    fmin = jnp.where(le, lo, hi)
    pmax = jnp.where(le, hi, lo)
    gidx = jnp.where(le, iota2, iota2 + MH)
    pidx = jnp.where(le, iota2 + MH, iota2)
    for k in range(K):
        mval = jnp.min(fmin, axis=1, keepdims=True)
        asel = jnp.min(jnp.where(fmin == mval, gidx, M), axis=1,
                       keepdims=True)
        idx_ref[0, :, k:k + 1] = asel + base
        valid_ref[0, :, k:k + 1] = jnp.where(mval < jnp.inf, 1.0, 0.0)
        if k + 1 < K:
            hit = gidx == asel
            fmin = jnp.where(hit, pmax, fmin)
            gidx = jnp.where(hit, pidx, gidx)
            pmax = jnp.where(hit, jnp.inf, pmax)


def _topk(q_xyz, kxt, interpret=False):
    grid = (B, N // _TN)
    return pl.pallas_call(
        _topk_body,
        grid=grid,
        in_specs=[
            pl.BlockSpec((1, _TN, 3), lambda b, i: (b, i, 0)),
            pl.BlockSpec((1, 3, M), lambda b, i: (b, 0, 0)),
        ],
        out_specs=[
            pl.BlockSpec((1, _TN, K), lambda b, i: (b, i, 0)),
            pl.BlockSpec((1, _TN, K), lambda b, i: (b, i, 0)),
        ],
        out_shape=[
            jax.ShapeDtypeStruct((B, N, K), jnp.int32),
            jax.ShapeDtypeStruct((B, N, K), jnp.float32),
        ],
        interpret=interpret,
    )(q_xyz, kxt)


def _gather_sc(ktab, vtab, idx_flat):
    info = plsc.get_sparse_core_info()
    nw = info.num_cores * info.num_subcores
    R = idx_flat.shape[0]
    D = ktab.shape[1]
    rows_per_w = R // nw
    CH = 128
    nch = rows_per_w // CH
    mesh = plsc.VectorSubcoreMesh(core_axis_name="c", subcore_axis_name="s")

    @functools.partial(
        pl.kernel, mesh=mesh,
        out_type=[
            jax.ShapeDtypeStruct((R, D), jnp.float32),
            jax.ShapeDtypeStruct((R, D), jnp.float32),
        ],
        scratch_types=[
            pltpu.VMEM((CH,), jnp.int32),
            pltpu.VMEM((CH, D), jnp.float32),
            pltpu.VMEM((CH, D), jnp.float32),
            pltpu.SemaphoreType.DMA,
            pltpu.SemaphoreType.DMA,
        ],
    )
    def gk(ktab_hbm, vtab_hbm, idx_hbm, kout_hbm, vout_hbm, idx_c, rows_k,
           rows_v, semk, semv):
        wid = jax.lax.axis_index("s") * info.num_cores + jax.lax.axis_index("c")
        base = wid * rows_per_w

        def body(i, carry):
            off = base + i * CH
            pltpu.sync_copy(idx_hbm.at[pl.ds(off, CH)], idx_c)
            ck = pltpu.async_copy(ktab_hbm.at[idx_c], rows_k, semk)
            cv = pltpu.async_copy(vtab_hbm.at[idx_c], rows_v, semv)
            ck.wait()
            pltpu.sync_copy(rows_k, kout_hbm.at[pl.ds(off, CH)])
            cv.wait()
            pltpu.sync_copy(rows_v, vout_hbm.at[pl.ds(off, CH)])
            return carry

        jax.lax.fori_loop(0, nch, body, 0)

    return gk(ktab, vtab, idx_flat)


_G = 256


def _attn_body(qf_ref, qx_ref, kgf_ref, vgf_ref, m128_ref, m0_ref,
               hsum_ref, hb0_ref, dn_ref, dn0_ref, br_ref, br0_ref,
               wq_ref, wk_ref, wv_ref,
               wpq_ref, wpk_ref, cq_ref, ck_ref, bv_ref, wo_ref, bo_ref,
               g1_ref, be1_ref, g2_ref, be2_ref, w1_ref, b1_ref, w2_ref,
               b2_ref, out_ref):
    f32 = jnp.float32
    qf = qf_ref[...]
    qx = qx_ref[...]
    q0 = (jnp.dot(qf, wq_ref[...], preferred_element_type=f32)
          + jnp.dot(qx, wpq_ref[...], preferred_element_type=f32) + cq_ref[...])
    k0 = (jnp.dot(qf, wk_ref[...], preferred_element_type=f32)
          + jnp.dot(qx, wpk_ref[...], preferred_element_type=f32) + ck_ref[...])
    v0 = jnp.dot(qf, wv_ref[...], preferred_element_type=f32) + bv_ref[...]
    kgf = kgf_ref[...]
    vgf = vgf_ref[...]
    scale = np.float32(1.0 / np.sqrt(DH))
    qrep = jnp.concatenate([q0] * K, axis=1)
    pk = qrep * kgf
    s = jnp.dot(pk, hsum_ref[...], preferred_element_type=f32) * scale
    s = jnp.where(m128_ref[...] > 0.0, s, -1e9)
    s0 = jnp.dot(q0 * k0, hb0_ref[...], preferred_element_type=f32) * scale
    s0 = jnp.where(m0_ref[...] > 0.0, s0, -1e9)
    mx = jnp.max(jnp.maximum(s, s0), axis=1, keepdims=True)
    e = jnp.exp(s - mx)
    e0 = jnp.exp(s0 - mx)
    den = (jnp.dot(e, dn_ref[...], preferred_element_type=f32)
           + jnp.dot(e0, dn0_ref[...], preferred_element_type=f32))
    w = e / den
    w0 = e0 / den
    wbr = jnp.dot(w, br_ref[...], preferred_element_type=f32)
    wv = wbr * vgf
    out0 = jnp.dot(w0, br0_ref[...], preferred_element_type=f32) * v0
    for j in range(K):
        out0 = out0 + wv[:, j * C:(j + 1) * C]
    y = jnp.dot(out0, wo_ref[...], preferred_element_type=f32) + bo_ref[...]

    x = qf + y
    mu = jnp.mean(x, axis=-1, keepdims=True)
    var = jnp.mean((x - mu) ** 2, axis=-1, keepdims=True)
    x = (x - mu) / jnp.sqrt(var + 1e-5) * g1_ref[...] + be1_ref[...]

    h1 = jnp.maximum(jnp.dot(x, w1_ref[...], preferred_element_type=f32)
                     + b1_ref[...], 0.0)
    ffv = jnp.dot(h1, w2_ref[...], preferred_element_type=f32) + b2_ref[...]

    x2 = x + ffv
    mu2 = jnp.mean(x2, axis=-1, keepdims=True)
    var2 = jnp.mean((x2 - mu2) ** 2, axis=-1, keepdims=True)
    x2 = (x2 - mu2) / jnp.sqrt(var2 + 1e-5) * g2_ref[...] + be2_ref[...]

    out_ref[...] = x2 + qf


def _mk_consts():
    l = np.arange(128)
    jj, hh = l // 16, l % 16
    r2 = np.arange(K * C)
    HSUM = np.zeros((K * C, 128), np.float32)
    HSUM[r2, (r2 // C) * 16 + (r2 % C) // DH] = 1.0
    r1 = np.arange(C)
    HB0 = np.zeros((C, 128), np.float32)
    HB0[r1, r1 // DH] = 1.0
    DN = ((hh[:, None] < 8) & (hh[None, :] == hh[:, None])).astype(np.float32)
    DN0 = np.zeros((128, 128), np.float32)
    for h in range(H):
        DN0[h, (hh == h) | (hh >= 8)] = 1.0
    BR = np.zeros((128, K * C), np.float32)
    mask_l = hh < 8
    BR[l[mask_l][:, None],
       (jj[mask_l] * C + hh[mask_l] * DH)[:, None] + np.arange(DH)[None, :]] = 1.0
    BR0 = np.zeros((128, C), np.float32)
    BR0[l[l < H][:, None], (l[l < H] * DH)[:, None] + np.arange(DH)[None, :]] = 1.0
    M0 = (l < H).astype(np.float32).reshape(1, 128)
    PAT = mask_l.astype(np.float32).reshape(1, 128)
    return HSUM, HB0, DN, DN0, BR, BR0, M0, PAT


def _attn(qf2, qx2, kgf, vgf, m128, m0, HSUM, HB0, DN, DN0, BR, BR0,
          Wq, Wk, Wv, Wpq, Wpk, cq, ck, bv, Wo, bo,
          g1, be1, g2, be2, W1, b1, W2, b2, interpret=False):
    grid = (BN // _G,)
    full = lambda r, c: pl.BlockSpec((r, c), lambda i: (0, 0))
    return pl.pallas_call(
        _attn_body,
        grid=grid,
        in_specs=[
            pl.BlockSpec((_G, C), lambda i: (i, 0)),
            pl.BlockSpec((_G, 3), lambda i: (i, 0)),
            pl.BlockSpec((_G, K * C), lambda i: (i, 0)),
            pl.BlockSpec((_G, K * C), lambda i: (i, 0)),
            pl.BlockSpec((_G, 128), lambda i: (i, 0)),
            full(1, 128),
            full(K * C, 128), full(C, 128), full(128, 128), full(128, 128),
            full(128, K * C), full(128, C),
            full(C, C), full(C, C), full(C, C), full(3, C), full(3, C),
            full(1, C), full(1, C), full(1, C), full(C, C), full(1, C),
            full(1, C), full(1, C), full(1, C), full(1, C),
            full(C, FF), full(1, FF), full(FF, C), full(1, C),
        ],
        out_specs=pl.BlockSpec((_G, C), lambda i: (i, 0)),
        out_shape=jax.ShapeDtypeStruct((BN, C), jnp.float32),
        interpret=interpret,
    )(qf2, qx2, kgf, vgf, m128, m0, HSUM, HB0, DN, DN0, BR, BR0,
      Wq, Wk, Wv, Wpq, Wpk, cq, ck, bv, Wo, bo,
      g1, be1, g2, be2, W1, b1, W2, b2)


def kernel(q_xyz, q_feat, kv_xyz, kv_feat, Wp, bp, Wq, bq, Wk, bk, Wv, bv,
           Wo, bo, g1, be1, g2, be2, W1, b1, W2, b2):
    Wpk = Wp @ Wk
    Wpq = Wp @ Wq
    ck = (bp @ Wk + bk).reshape(1, C)
    cq = (bp @ Wq + bq).reshape(1, C)
    bv2 = bv.reshape(1, C)

    kvf2 = kv_feat.reshape(BM, C)
    kvx2 = kv_xyz.reshape(BM, 3)
    kxt = kv_xyz.transpose(0, 2, 1)

    ktab, vtab = _proj_kv(kvf2, kvx2, Wk, Wv, Wpk, ck, bv2)
    idxg, valid = _topk(q_xyz, kxt)

    gk, gv = _gather_sc(ktab, vtab, idxg.reshape(BN * K))
    kgf = gk.reshape(BN, K * C)
    vgf = gv.reshape(BN, K * C)

    HSUM, HB0, DN, DN0, BR, BR0, M0, PAT = _mk_consts()
    m128 = jnp.repeat(valid.reshape(BN, K), 16, axis=1) * PAT

    out = _attn(q_feat.reshape(BN, C), q_xyz.reshape(BN, 3), kgf, vgf,
                m128, M0, HSUM, HB0, DN, DN0, BR, BR0,
                Wq, Wk, Wv, Wpq, Wpk, cq, ck, bv2,
                Wo, bo.reshape(1, C), g1.reshape(1, C), be1.reshape(1, C),
                g2.reshape(1, C), be2.reshape(1, C), W1, b1.reshape(1, FF),
                W2, b2.reshape(1, C))
    return out.reshape(B, N, C)


# speedup vs baseline: 1.0207x; 1.0207x over previous
import functools

import jax
import jax.numpy as jnp
import numpy as np
from jax.experimental import pallas as pl
from jax.experimental.pallas import tpu as pltpu
from jax.experimental.pallas import tpu_sc as plsc

B, N, M, C, H, K = 4, 2048, 4096, 256, 8, 8
RADIUS = 0.2
DH = C // H
FF = 4 * C
KP = K + 1
BN = B * N
BM = B * M


def _proj_kv_body(kvf_ref, kvx_ref, wk_ref, wv_ref, wpk_ref, ck_ref, bv_ref,
                  kout_ref, vout_ref):
    kvf = kvf_ref[...]
    kvx = kvx_ref[...]
    kout_ref[...] = (jnp.dot(kvf, wk_ref[...], preferred_element_type=jnp.float32)
                     + jnp.dot(kvx, wpk_ref[...], preferred_element_type=jnp.float32)
                     + ck_ref[...])
    vout_ref[...] = (jnp.dot(kvf, wv_ref[...], preferred_element_type=jnp.float32)
                     + bv_ref[...])


def _proj_kv(kvf2, kvx2, Wk, Wv, Wpk, ck, bv, interpret=False):
    T = 2048
    grid = (BM // T,)
    return pl.pallas_call(
        _proj_kv_body,
        grid=grid,
        in_specs=[
            pl.BlockSpec((T, C), lambda i: (i, 0)),
            pl.BlockSpec((T, 3), lambda i: (i, 0)),
            pl.BlockSpec((C, C), lambda i: (0, 0)),
            pl.BlockSpec((C, C), lambda i: (0, 0)),
            pl.BlockSpec((3, C), lambda i: (0, 0)),
            pl.BlockSpec((1, C), lambda i: (0, 0)),
            pl.BlockSpec((1, C), lambda i: (0, 0)),
        ],
        out_specs=[
            pl.BlockSpec((T, C), lambda i: (i, 0)),
            pl.BlockSpec((T, C), lambda i: (i, 0)),
        ],
        out_shape=[
            jax.ShapeDtypeStruct((BM, C), jnp.float32),
            jax.ShapeDtypeStruct((BM, C), jnp.float32),
        ],
        interpret=interpret,
    )(kvf2, kvx2, Wk, Wv, Wpk, ck, bv)


_TN = 256


def _topk_body(qx_ref, kxt_ref, idx_ref, valid_ref):
    b = pl.program_id(0)
    qx = qx_ref[0]
    kxt = kxt_ref[0]
    qn = jnp.sum(qx * qx, axis=1, keepdims=True)
    kn = jnp.sum(kxt * kxt, axis=0, keepdims=True)
    dot = jnp.dot(qx.astype(jnp.bfloat16), kxt.astype(jnp.bfloat16),
                  preferred_element_type=jnp.float32)
    d2 = qn + kn - 2.0 * dot
    dist = jnp.sqrt(jnp.maximum(d2, 1e-12))
    work = jnp.where(dist <= RADIUS, dist, jnp.inf)
    base = b * M
    MH = M // 2
    lo = work[:, :MH]
    hi = work[:, MH:]
    le = lo <= hi
    iota2 = jax.lax.broadcasted_iota(jnp.int32, (_TN, MH), 1)
    fmin = jnp.where(le, lo, hi)
    pmax = jnp.where(le, hi, lo)
    gidx = jnp.where(le, iota2, iota2 + MH)
    pidx = jnp.where(le, iota2 + MH, iota2)
    for k in range(K):
        mval = jnp.min(fmin, axis=1, keepdims=True)
        asel = jnp.min(jnp.where(fmin == mval, gidx, M), axis=1,
                       keepdims=True)
        idx_ref[0, :, k:k + 1] = asel + base
        valid_ref[0, :, k:k + 1] = jnp.where(mval < jnp.inf, 1.0, 0.0)
        if k + 1 < K:
            hit = gidx == asel
            fmin = jnp.where(hit, pmax, fmin)
            gidx = jnp.where(hit, pidx, gidx)
            pmax = jnp.where(hit, jnp.inf, pmax)


def _topk(q_xyz, kxt, interpret=False):
    grid = (B, N // _TN)
    return pl.pallas_call(
        _topk_body,
        grid=grid,
        in_specs=[
            pl.BlockSpec((1, _TN, 3), lambda b, i: (b, i, 0)),
            pl.BlockSpec((1, 3, M), lambda b, i: (b, 0, 0)),
        ],
        out_specs=[
            pl.BlockSpec((1, _TN, K), lambda b, i: (b, i, 0)),
            pl.BlockSpec((1, _TN, K), lambda b, i: (b, i, 0)),
        ],
        out_shape=[
            jax.ShapeDtypeStruct((B, N, K), jnp.int32),
            jax.ShapeDtypeStruct((B, N, K), jnp.float32),
        ],
        interpret=interpret,
    )(q_xyz, kxt)


def _gather_sc(ktab, vtab, idx_flat):
    info = plsc.get_sparse_core_info()
    nw = info.num_cores * info.num_subcores
    R = idx_flat.shape[0]
    D = ktab.shape[1]
    rows_per_w = R // nw
    CH = 128
    nch = rows_per_w // CH
    mesh = plsc.VectorSubcoreMesh(core_axis_name="c", subcore_axis_name="s")

    @functools.partial(
        pl.kernel, mesh=mesh,
        out_type=[
            jax.ShapeDtypeStruct((R, D), jnp.float32),
            jax.ShapeDtypeStruct((R, D), jnp.float32),
        ],
        scratch_types=[
            pltpu.VMEM((CH,), jnp.int32),
            pltpu.VMEM((CH, D), jnp.float32),
            pltpu.VMEM((CH, D), jnp.float32),
            pltpu.SemaphoreType.DMA,
            pltpu.SemaphoreType.DMA,
        ],
    )
    def gk(ktab_hbm, vtab_hbm, idx_hbm, kout_hbm, vout_hbm, idx_c, rows_k,
           rows_v, semk, semv):
        wid = jax.lax.axis_index("s") * info.num_cores + jax.lax.axis_index("c")
        base = wid * rows_per_w

        def body(i, carry):
            off = base + i * CH
            pltpu.sync_copy(idx_hbm.at[pl.ds(off, CH)], idx_c)
            ck = pltpu.async_copy(ktab_hbm.at[idx_c], rows_k, semk)
            cv = pltpu.async_copy(vtab_hbm.at[idx_c], rows_v, semv)
            ck.wait()
            pltpu.sync_copy(rows_k, kout_hbm.at[pl.ds(off, CH)])
            cv.wait()
            pltpu.sync_copy(rows_v, vout_hbm.at[pl.ds(off, CH)])
            return carry

        jax.lax.fori_loop(0, nch, body, 0)

    return gk(ktab, vtab, idx_flat)


_G = 256


def _attn_body(qf_ref, qx_ref, kgf_ref, vgf_ref, m128_ref, m0_ref,
               hsum_ref, hb0_ref, dn_ref, dn0_ref, br_ref, br0_ref,
               wq_ref, wk_ref, wv_ref,
               wpq_ref, wpk_ref, cq_ref, ck_ref, bv_ref, wo_ref, bo_ref,
               g1_ref, be1_ref, g2_ref, be2_ref, w1_ref, b1_ref, w2_ref,
               b2_ref, out_ref):
    f32 = jnp.float32
    qf = qf_ref[...]
    qx = qx_ref[...]
    q0 = (jnp.dot(qf, wq_ref[...], preferred_element_type=f32)
          + jnp.dot(qx, wpq_ref[...], preferred_element_type=f32) + cq_ref[...])
    k0 = (jnp.dot(qf, wk_ref[...], preferred_element_type=f32)
          + jnp.dot(qx, wpk_ref[...], preferred_element_type=f32) + ck_ref[...])
    v0 = jnp.dot(qf, wv_ref[...], preferred_element_type=f32) + bv_ref[...]
    kgf = kgf_ref[...]
    vgf = vgf_ref[...]
    scale = np.float32(1.0 / np.sqrt(DH))
    qrep = jnp.concatenate([q0] * K, axis=1)
    pk = qrep * kgf
    s = jnp.dot(pk, hsum_ref[...], preferred_element_type=f32) * scale
    s = jnp.where(m128_ref[...] > 0.0, s, -1e9)
    s0 = jnp.dot(q0 * k0, hb0_ref[...], preferred_element_type=f32) * scale
    s0 = jnp.where(m0_ref[...] > 0.0, s0, -1e9)
    mx = jnp.max(jnp.maximum(s, s0), axis=1, keepdims=True)
    e = jnp.exp(s - mx)
    e0 = jnp.exp(s0 - mx)
    den = (jnp.dot(e, dn_ref[...], preferred_element_type=f32)
           + jnp.dot(e0, dn0_ref[...], preferred_element_type=f32))
    w = e / den
    w0 = e0 / den
    wbr = jnp.dot(w, br_ref[...], preferred_element_type=f32)
    wv = wbr * vgf
    out0 = jnp.dot(w0, br0_ref[...], preferred_element_type=f32) * v0
    for j in range(K):
        out0 = out0 + wv[:, j * C:(j + 1) * C]
    y = jnp.dot(out0, wo_ref[...], preferred_element_type=f32) + bo_ref[...]

    x = qf + y
    mu = jnp.mean(x, axis=-1, keepdims=True)
    var = jnp.mean((x - mu) ** 2, axis=-1, keepdims=True)
    x = (x - mu) / jnp.sqrt(var + 1e-5) * g1_ref[...] + be1_ref[...]

    h1 = jnp.maximum(jnp.dot(x, w1_ref[...], preferred_element_type=f32)
                     + b1_ref[...], 0.0)
    ffv = jnp.dot(h1, w2_ref[...], preferred_element_type=f32) + b2_ref[...]

    x2 = x + ffv
    mu2 = jnp.mean(x2, axis=-1, keepdims=True)
    var2 = jnp.mean((x2 - mu2) ** 2, axis=-1, keepdims=True)
    x2 = (x2 - mu2) / jnp.sqrt(var2 + 1e-5) * g2_ref[...] + be2_ref[...]

    out_ref[...] = x2 + qf


def _mk_consts():
    l = np.arange(128)
    jj, hh = l // 16, l % 16
    r2 = np.arange(K * C)
    HSUM = np.zeros((K * C, 128), np.float32)
    HSUM[r2, (r2 // C) * 16 + (r2 % C) // DH] = 1.0
    r1 = np.arange(C)
    HB0 = np.zeros((C, 128), np.float32)
    HB0[r1, r1 // DH] = 1.0
    DN = ((hh[:, None] < 8) & (hh[None, :] == hh[:, None])).astype(np.float32)
    DN0 = np.zeros((128, 128), np.float32)
    for h in range(H):
        DN0[h, (hh == h) | (hh >= 8)] = 1.0
    BR = np.zeros((128, K * C), np.float32)
    mask_l = hh < 8
    BR[l[mask_l][:, None],
       (jj[mask_l] * C + hh[mask_l] * DH)[:, None] + np.arange(DH)[None, :]] = 1.0
    BR0 = np.zeros((128, C), np.float32)
    BR0[l[l < H][:, None], (l[l < H] * DH)[:, None] + np.arange(DH)[None, :]] = 1.0
    M0 = (l < H).astype(np.float32).reshape(1, 128)
    PAT = mask_l.astype(np.float32).reshape(1, 128)
    return HSUM, HB0, DN, DN0, BR, BR0, M0, PAT


def _attn(qf2, qx2, kgf, vgf, m128, m0, HSUM, HB0, DN, DN0, BR, BR0,
          Wq, Wk, Wv, Wpq, Wpk, cq, ck, bv, Wo, bo,
          g1, be1, g2, be2, W1, b1, W2, b2, interpret=False):
    rows = qf2.shape[0]
    grid = (rows // _G,)
    full = lambda r, c: pl.BlockSpec((r, c), lambda i: (0, 0))
    return pl.pallas_call(
        _attn_body,
        grid=grid,
        in_specs=[
            pl.BlockSpec((_G, C), lambda i: (i, 0)),
            pl.BlockSpec((_G, 3), lambda i: (i, 0)),
            pl.BlockSpec((_G, K * C), lambda i: (i, 0)),
            pl.BlockSpec((_G, K * C), lambda i: (i, 0)),
            pl.BlockSpec((_G, 128), lambda i: (i, 0)),
            full(1, 128),
            full(K * C, 128), full(C, 128), full(128, 128), full(128, 128),
            full(128, K * C), full(128, C),
            full(C, C), full(C, C), full(C, C), full(3, C), full(3, C),
            full(1, C), full(1, C), full(1, C), full(C, C), full(1, C),
            full(1, C), full(1, C), full(1, C), full(1, C),
            full(C, FF), full(1, FF), full(FF, C), full(1, C),
        ],
        out_specs=pl.BlockSpec((_G, C), lambda i: (i, 0)),
        out_shape=jax.ShapeDtypeStruct((rows, C), jnp.float32),
        interpret=interpret,
    )(qf2, qx2, kgf, vgf, m128, m0, HSUM, HB0, DN, DN0, BR, BR0,
      Wq, Wk, Wv, Wpq, Wpk, cq, ck, bv, Wo, bo,
      g1, be1, g2, be2, W1, b1, W2, b2)


def kernel(q_xyz, q_feat, kv_xyz, kv_feat, Wp, bp, Wq, bq, Wk, bk, Wv, bv,
           Wo, bo, g1, be1, g2, be2, W1, b1, W2, b2):
    Wpk = Wp @ Wk
    Wpq = Wp @ Wq
    ck = (bp @ Wk + bk).reshape(1, C)
    cq = (bp @ Wq + bq).reshape(1, C)
    bv2 = bv.reshape(1, C)

    kvf2 = kv_feat.reshape(BM, C)
    kvx2 = kv_xyz.reshape(BM, 3)
    kxt = kv_xyz.transpose(0, 2, 1)

    ktab, vtab = _proj_kv(kvf2, kvx2, Wk, Wv, Wpk, ck, bv2)
    idxg, valid = _topk(q_xyz, kxt)

    HSUM, HB0, DN, DN0, BR, BR0, M0, PAT = _mk_consts()
    m128 = jnp.repeat(valid.reshape(BN, K), 16, axis=1) * PAT
    idx_flat = idxg.reshape(BN * K)
    qf2 = q_feat.reshape(BN, C)
    qx2 = q_xyz.reshape(BN, 3)

    HR = BN // 2
    outs = []
    gathered = [_gather_sc(ktab, vtab, idx_flat[i * HR * K:(i + 1) * HR * K])
                for i in range(2)]
    for i, (gk, gv) in enumerate(gathered):
        sl = slice(i * HR, (i + 1) * HR)
        outs.append(_attn(qf2[sl], qx2[sl], gk.reshape(HR, K * C),
                          gv.reshape(HR, K * C), m128[sl], M0,
                          HSUM, HB0, DN, DN0, BR, BR0,
                          Wq, Wk, Wv, Wpq, Wpk, cq, ck, bv2,
                          Wo, bo.reshape(1, C), g1.reshape(1, C),
                          be1.reshape(1, C), g2.reshape(1, C),
                          be2.reshape(1, C), W1, b1.reshape(1, FF),
                          W2, b2.reshape(1, C)))
    return jnp.concatenate(outs, axis=0).reshape(B, N, C)


# speedup vs baseline: 1.0274x; 1.0066x over previous
import functools

import jax
import jax.numpy as jnp
import numpy as np
from jax.experimental import pallas as pl
from jax.experimental.pallas import tpu as pltpu
from jax.experimental.pallas import tpu_sc as plsc

B, N, M, C, H, K = 4, 2048, 4096, 256, 8, 8
RADIUS = 0.2
DH = C // H
FF = 4 * C
KP = K + 1
BN = B * N
BM = B * M


def _proj_kv_body(kvf_ref, kvx_ref, wk_ref, wv_ref, wpk_ref, ck_ref, bv_ref,
                  kout_ref, vout_ref):
    kvf = kvf_ref[...]
    kvx = kvx_ref[...]
    kout_ref[...] = (jnp.dot(kvf, wk_ref[...], preferred_element_type=jnp.float32)
                     + jnp.dot(kvx, wpk_ref[...], preferred_element_type=jnp.float32)
                     + ck_ref[...])
    vout_ref[...] = (jnp.dot(kvf, wv_ref[...], preferred_element_type=jnp.float32)
                     + bv_ref[...])


def _proj_kv(kvf2, kvx2, Wk, Wv, Wpk, ck, bv, interpret=False):
    T = 2048
    grid = (BM // T,)
    return pl.pallas_call(
        _proj_kv_body,
        grid=grid,
        in_specs=[
            pl.BlockSpec((T, C), lambda i: (i, 0)),
            pl.BlockSpec((T, 3), lambda i: (i, 0)),
            pl.BlockSpec((C, C), lambda i: (0, 0)),
            pl.BlockSpec((C, C), lambda i: (0, 0)),
            pl.BlockSpec((3, C), lambda i: (0, 0)),
            pl.BlockSpec((1, C), lambda i: (0, 0)),
            pl.BlockSpec((1, C), lambda i: (0, 0)),
        ],
        out_specs=[
            pl.BlockSpec((T, C), lambda i: (i, 0)),
            pl.BlockSpec((T, C), lambda i: (i, 0)),
        ],
        out_shape=[
            jax.ShapeDtypeStruct((BM, C), jnp.float32),
            jax.ShapeDtypeStruct((BM, C), jnp.float32),
        ],
        interpret=interpret,
    )(kvf2, kvx2, Wk, Wv, Wpk, ck, bv)


_TN = 256


def _topk_body(qx_ref, kxt_ref, idx_ref, valid_ref):
    b = pl.program_id(0)
    qx = qx_ref[0]
    kxt = kxt_ref[0]
    qn = jnp.sum(qx * qx, axis=1, keepdims=True)
    kn = jnp.sum(kxt * kxt, axis=0, keepdims=True)
    dot = jnp.dot(qx.astype(jnp.bfloat16), kxt.astype(jnp.bfloat16),
                  preferred_element_type=jnp.float32)
    d2 = qn + kn - 2.0 * dot
    dist = jnp.sqrt(jnp.maximum(d2, 1e-12))
    work = jnp.where(dist <= RADIUS, dist, jnp.inf)
    base = b * M
    MH = M // 2
    lo = work[:, :MH]
    hi = work[:, MH:]
    le = lo <= hi
    iota2 = jax.lax.broadcasted_iota(jnp.int32, (_TN, MH), 1)
    fmin = jnp.where(le, lo, hi)
    pmax = jnp.where(le, hi, lo)
    gidx = jnp.where(le, iota2, iota2 + MH)
    pidx = jnp.where(le, iota2 + MH, iota2)
    for k in range(K):
        mval = jnp.min(fmin, axis=1, keepdims=True)
        asel = jnp.min(jnp.where(fmin == mval, gidx, M), axis=1,
                       keepdims=True)
        idx_ref[0, :, k:k + 1] = asel + base
        valid_ref[0, :, k:k + 1] = jnp.where(mval < jnp.inf, 1.0, 0.0)
        if k + 1 < K:
            hit = gidx == asel
            fmin = jnp.where(hit, pmax, fmin)
            gidx = jnp.where(hit, pidx, gidx)
            pmax = jnp.where(hit, jnp.inf, pmax)


def _topk(q_xyz, kxt, interpret=False):
    grid = (B, N // _TN)
    return pl.pallas_call(
        _topk_body,
        grid=grid,
        in_specs=[
            pl.BlockSpec((1, _TN, 3), lambda b, i: (b, i, 0)),
            pl.BlockSpec((1, 3, M), lambda b, i: (b, 0, 0)),
        ],
        out_specs=[
            pl.BlockSpec((1, _TN, K), lambda b, i: (b, i, 0)),
            pl.BlockSpec((1, _TN, K), lambda b, i: (b, i, 0)),
        ],
        out_shape=[
            jax.ShapeDtypeStruct((B, N, K), jnp.int32),
            jax.ShapeDtypeStruct((B, N, K), jnp.float32),
        ],
        interpret=interpret,
    )(q_xyz, kxt)


def _gather_sc(ktab, vtab, idx_flat):
    info = plsc.get_sparse_core_info()
    nw = info.num_cores * info.num_subcores
    R = idx_flat.shape[0]
    D = ktab.shape[1]
    rows_per_w = R // nw
    CH = 128
    nch = rows_per_w // CH
    mesh = plsc.VectorSubcoreMesh(core_axis_name="c", subcore_axis_name="s")

    @functools.partial(
        pl.kernel, mesh=mesh,
        out_type=[
            jax.ShapeDtypeStruct((R, D), jnp.float32),
            jax.ShapeDtypeStruct((R, D), jnp.float32),
        ],
        scratch_types=[
            pltpu.VMEM((CH,), jnp.int32),
            pltpu.VMEM((CH, D), jnp.float32),
            pltpu.VMEM((CH, D), jnp.float32),
            pltpu.SemaphoreType.DMA,
            pltpu.SemaphoreType.DMA,
        ],
    )
    def gk(ktab_hbm, vtab_hbm, idx_hbm, kout_hbm, vout_hbm, idx_c, rows_k,
           rows_v, semk, semv):
        wid = jax.lax.axis_index("s") * info.num_cores + jax.lax.axis_index("c")
        base = wid * rows_per_w

        def body(i, carry):
            off = base + i * CH
            pltpu.sync_copy(idx_hbm.at[pl.ds(off, CH)], idx_c)
            ck = pltpu.async_copy(ktab_hbm.at[idx_c], rows_k, semk)
            cv = pltpu.async_copy(vtab_hbm.at[idx_c], rows_v, semv)
            ck.wait()
            pltpu.sync_copy(rows_k, kout_hbm.at[pl.ds(off, CH)])
            cv.wait()
            pltpu.sync_copy(rows_v, vout_hbm.at[pl.ds(off, CH)])
            return carry

        jax.lax.fori_loop(0, nch, body, 0)

    return gk(ktab, vtab, idx_flat)


_G = 256


def _attn_body(qf_ref, qx_ref, kgf_ref, vgf_ref, m128_ref, m0_ref,
               hsum_ref, hb0_ref, dn_ref, dn0_ref, br_ref, br0_ref,
               wq_ref, wk_ref, wv_ref,
               wpq_ref, wpk_ref, cq_ref, ck_ref, bv_ref, wo_ref, bo_ref,
               g1_ref, be1_ref, g2_ref, be2_ref, w1_ref, b1_ref, w2_ref,
               b2_ref, out_ref):
    f32 = jnp.float32
    qf = qf_ref[...]
    qx = qx_ref[...]
    q0 = (jnp.dot(qf, wq_ref[...], preferred_element_type=f32)
          + jnp.dot(qx, wpq_ref[...], preferred_element_type=f32) + cq_ref[...])
    k0 = (jnp.dot(qf, wk_ref[...], preferred_element_type=f32)
          + jnp.dot(qx, wpk_ref[...], preferred_element_type=f32) + ck_ref[...])
    v0 = jnp.dot(qf, wv_ref[...], preferred_element_type=f32) + bv_ref[...]
    kgf = kgf_ref[...]
    vgf = vgf_ref[...]
    scale = np.float32(1.0 / np.sqrt(DH))
    qrep = jnp.concatenate([q0] * K, axis=1)
    pk = qrep * kgf
    s = jnp.dot(pk, hsum_ref[...], preferred_element_type=f32) * scale
    s = jnp.where(m128_ref[...] > 0.0, s, -1e9)
    s0 = jnp.dot(q0 * k0, hb0_ref[...], preferred_element_type=f32) * scale
    s0 = jnp.where(m0_ref[...] > 0.0, s0, -1e9)
    mx = jnp.max(jnp.maximum(s, s0), axis=1, keepdims=True)
    e = jnp.exp(s - mx)
    e0 = jnp.exp(s0 - mx)
    den = (jnp.dot(e, dn_ref[...], preferred_element_type=f32)
           + jnp.dot(e0, dn0_ref[...], preferred_element_type=f32))
    w = e / den
    w0 = e0 / den
    wbr = jnp.dot(w, br_ref[...], preferred_element_type=f32)
    wv = wbr * vgf
    out0 = jnp.dot(w0, br0_ref[...], preferred_element_type=f32) * v0
    for j in range(K):
        out0 = out0 + wv[:, j * C:(j + 1) * C]
    y = jnp.dot(out0, wo_ref[...], preferred_element_type=f32) + bo_ref[...]

    x = qf + y
    mu = jnp.mean(x, axis=-1, keepdims=True)
    var = jnp.mean((x - mu) ** 2, axis=-1, keepdims=True)
    x = (x - mu) / jnp.sqrt(var + 1e-5) * g1_ref[...] + be1_ref[...]

    h1 = jnp.maximum(jnp.dot(x, w1_ref[...], preferred_element_type=f32)
                     + b1_ref[...], 0.0)
    ffv = jnp.dot(h1, w2_ref[...], preferred_element_type=f32) + b2_ref[...]

    x2 = x + ffv
    mu2 = jnp.mean(x2, axis=-1, keepdims=True)
    var2 = jnp.mean((x2 - mu2) ** 2, axis=-1, keepdims=True)
    x2 = (x2 - mu2) / jnp.sqrt(var2 + 1e-5) * g2_ref[...] + be2_ref[...]

    out_ref[...] = x2 + qf


def _mk_consts():
    l = np.arange(128)
    jj, hh = l // 16, l % 16
    r2 = np.arange(K * C)
    HSUM = np.zeros((K * C, 128), np.float32)
    HSUM[r2, (r2 // C) * 16 + (r2 % C) // DH] = 1.0
    r1 = np.arange(C)
    HB0 = np.zeros((C, 128), np.float32)
    HB0[r1, r1 // DH] = 1.0
    DN = ((hh[:, None] < 8) & (hh[None, :] == hh[:, None])).astype(np.float32)
    DN0 = np.zeros((128, 128), np.float32)
    for h in range(H):
        DN0[h, (hh == h) | (hh >= 8)] = 1.0
    BR = np.zeros((128, K * C), np.float32)
    mask_l = hh < 8
    BR[l[mask_l][:, None],
       (jj[mask_l] * C + hh[mask_l] * DH)[:, None] + np.arange(DH)[None, :]] = 1.0
    BR0 = np.zeros((128, C), np.float32)
    BR0[l[l < H][:, None], (l[l < H] * DH)[:, None] + np.arange(DH)[None, :]] = 1.0
    M0 = (l < H).astype(np.float32).reshape(1, 128)
    PAT = mask_l.astype(np.float32).reshape(1, 128)
    return HSUM, HB0, DN, DN0, BR, BR0, M0, PAT


def _attn(qf2, qx2, kgf, vgf, m128, m0, HSUM, HB0, DN, DN0, BR, BR0,
          Wq, Wk, Wv, Wpq, Wpk, cq, ck, bv, Wo, bo,
          g1, be1, g2, be2, W1, b1, W2, b2, interpret=False):
    rows = qf2.shape[0]
    grid = (rows // _G,)
    full = lambda r, c: pl.BlockSpec((r, c), lambda i: (0, 0))
    return pl.pallas_call(
        _attn_body,
        grid=grid,
        in_specs=[
            pl.BlockSpec((_G, C), lambda i: (i, 0)),
            pl.BlockSpec((_G, 3), lambda i: (i, 0)),
            pl.BlockSpec((_G, K * C), lambda i: (i, 0)),
            pl.BlockSpec((_G, K * C), lambda i: (i, 0)),
            pl.BlockSpec((_G, 128), lambda i: (i, 0)),
            full(1, 128),
            full(K * C, 128), full(C, 128), full(128, 128), full(128, 128),
            full(128, K * C), full(128, C),
            full(C, C), full(C, C), full(C, C), full(3, C), full(3, C),
            full(1, C), full(1, C), full(1, C), full(C, C), full(1, C),
            full(1, C), full(1, C), full(1, C), full(1, C),
            full(C, FF), full(1, FF), full(FF, C), full(1, C),
        ],
        out_specs=pl.BlockSpec((_G, C), lambda i: (i, 0)),
        out_shape=jax.ShapeDtypeStruct((rows, C), jnp.float32),
        interpret=interpret,
    )(qf2, qx2, kgf, vgf, m128, m0, HSUM, HB0, DN, DN0, BR, BR0,
      Wq, Wk, Wv, Wpq, Wpk, cq, ck, bv, Wo, bo,
      g1, be1, g2, be2, W1, b1, W2, b2)


def kernel(q_xyz, q_feat, kv_xyz, kv_feat, Wp, bp, Wq, bq, Wk, bk, Wv, bv,
           Wo, bo, g1, be1, g2, be2, W1, b1, W2, b2):
    Wpk = Wp @ Wk
    Wpq = Wp @ Wq
    ck = (bp @ Wk + bk).reshape(1, C)
    cq = (bp @ Wq + bq).reshape(1, C)
    bv2 = bv.reshape(1, C)

    kvf2 = kv_feat.reshape(BM, C)
    kvx2 = kv_xyz.reshape(BM, 3)
    kxt = kv_xyz.transpose(0, 2, 1)

    ktab, vtab = _proj_kv(kvf2, kvx2, Wk, Wv, Wpk, ck, bv2)
    idxg, valid = _topk(q_xyz, kxt)

    gk, gv = _gather_sc(ktab, vtab, idxg.reshape(BN * K))
    kgf = gk.reshape(BN, K * C)
    vgf = gv.reshape(BN, K * C)

    HSUM, HB0, DN, DN0, BR, BR0, M0, PAT = _mk_consts()
    m128 = jnp.repeat(valid.reshape(BN, K), 16, axis=1) * PAT

    out = _attn(q_feat.reshape(BN, C), q_xyz.reshape(BN, 3), kgf, vgf,
                m128, M0, HSUM, HB0, DN, DN0, BR, BR0,
                Wq, Wk, Wv, Wpq, Wpk, cq, ck, bv2,
                Wo, bo.reshape(1, C), g1.reshape(1, C), be1.reshape(1, C),
                g2.reshape(1, C), be2.reshape(1, C), W1, b1.reshape(1, FF),
                W2, b2.reshape(1, C))
    return out.reshape(B, N, C)
